# baseline (speedup 1.0000x reference)
import functools

import jax
import jax.numpy as jnp
import numpy as np
from jax.experimental import pallas as pl
from jax.experimental.pallas import tpu as pltpu
from jax.experimental.pallas import tpu_sc as plsc

_NB = 32
_CUT = 5.0
_C = 64
_NSH = 9
_HS = 64
_HV = 32
_DIMH = _HS + 3 * _HV
_NLAYERS = 3

_W = 256
_B = 256
_TE = 1024


def _sc_gather(table, idx):
    m = idx.shape[0]
    n, d = table.shape
    win = 256
    mesh = plsc.VectorSubcoreMesh(core_axis_name="c", subcore_axis_name="s")

    @functools.partial(
        pl.kernel, mesh=mesh,
        out_type=jax.ShapeDtypeStruct((m, d), table.dtype))
    def k(x_hbm, i_hbm, o_hbm):
        def body(i_vmem, o_vmem):
            pltpu.sync_copy(x_hbm.at[i_vmem.at[0]], o_vmem)

        pltpu.emit_pipeline(
            body,
            grid=(m // win,),
            in_specs=[pl.BlockSpec((1, win), lambda i: (0, i))],
            out_specs=[pl.BlockSpec((win, d), lambda i: (i, 0))],
            core_axis_name=("c", "s"),
            dimension_semantics=(pltpu.PARALLEL,),
        )(i_hbm, o_hbm)

    return k(table, idx.reshape(1, m))


def _geom_body(ps_ref, pd_ref, feat_ref):
    ps = ps_ref[:, 0:3]
    pd = pd_ref[:, 0:3]
    v = pd - ps
    r = jnp.sqrt(jnp.sum(v * v, axis=1, keepdims=True) + 1e-16)
    d = v / jnp.maximum(r, 1e-8)
    x = d[:, 0:1]
    y = d[:, 1:2]
    z = d[:, 2:3]
    c1 = np.float32(np.sqrt(3.0))
    c2 = np.float32(np.sqrt(15.0))
    c3 = np.float32(np.sqrt(5.0) / 2.0)
    sh = jnp.concatenate([
        jnp.ones_like(x), c1 * y, c1 * z, c1 * x,
        c2 * x * y, c2 * y * z, c3 * (3.0 * z * z - 1.0), c2 * x * z,
        (c2 / 2.0) * (x * x - y * y)
    ], axis=1)
    u = (r / _CUT) ** 2
    us = jnp.minimum(u, 0.99)
    env = jnp.where(r < _CUT, jnp.exp(1.0 - 1.0 / (1.0 - us)), 0.0)
    step = np.float32(_CUT / (_NB - 1))
    centers = (jax.lax.broadcasted_iota(jnp.int32, (1, _NB), 1)
               .astype(jnp.float32) * step)
    width = _CUT / _NB
    g = jnp.exp(-((r - centers) ** 2) / (2.0 * width * width))
    ea = g * env
    shg = sh * env
    pad = jnp.zeros((ps.shape[0], 64 - _NB - _NSH), dtype=jnp.float32)
    feat_ref[...] = jnp.concatenate([ea, shg, pad], axis=1)


def _mm_body(x_ref, w_ref, o_ref):
    o_ref[...] = jax.lax.dot_general(
        x_ref[...], w_ref[...], (((1,), (0,)), ((), ())),
        preferred_element_type=jnp.float32)


def _edge_agg_body(boc_ref, first_ref, feat_ref, hsrc_ref, dl_ref,
                   w1_ref, b1_ref, w2_ref, out_ref):
    j = pl.program_id(0)
    ea = feat_ref[:, 0:_NB]
    shg = feat_ref[:, _NB:_NB + _NSH]
    t = jax.lax.dot_general(ea, w1_ref[...], (((1,), (0,)), ((), ())),
                            preferred_element_type=jnp.float32)
    t = jnp.maximum(t + b1_ref[...], 0.0)
    R = jax.lax.dot_general(t, w2_ref[...], (((1,), (0,)), ((), ())),
                            preferred_element_type=jnp.float32)
    m = hsrc_ref[:, 0:_C] * R
    msh = jnp.concatenate(
        [m * shg[:, k:k + 1] for k in range(_NSH)], axis=1
    ).astype(jnp.bfloat16)
    dl = dl_ref[0, 0, :]
    oh = (jax.lax.broadcasted_iota(jnp.int32, (_W, _B), 0)
          == dl[None, :]).astype(jnp.bfloat16)
    contrib = jax.lax.dot_general(oh, msh, (((1,), (0,)), ((), ())),
                                  preferred_element_type=jnp.float32)
    first = first_ref[j]

    @pl.when(first == 1)
    def _():
        out_ref[...] = contrib

    @pl.when(first == 0)
    def _():
        out_ref[...] += contrib


def _node_out_body(agg_ref, x_ref, wout_ref, wself_ref, *rest, with_h):
    if with_h:
        wpre_ref, out_ref, h_ref = rest
    else:
        (out_ref,) = rest
    out = jax.lax.dot_general(agg_ref[...], wout_ref[...],
                              (((1,), (0,)), ((), ())),
                              preferred_element_type=jnp.float32)
    out += jax.lax.dot_general(x_ref[...], wself_ref[...],
                               (((1,), (0,)), ((), ())),
                               preferred_element_type=jnp.float32)
    out_ref[...] = out
    if with_h:
        h_ref[...] = jax.lax.dot_general(out, wpre_ref[...],
                                         (((1,), (0,)), ((), ())),
                                         preferred_element_type=jnp.float32)


def _matmul(x, w):
    n, d = x.shape
    _, o = w.shape
    return pl.pallas_call(
        _mm_body,
        grid=(n // _W,),
        in_specs=[pl.BlockSpec((_W, d), lambda i: (i, 0)),
                  pl.BlockSpec((d, o), lambda i: (0, 0))],
        out_specs=pl.BlockSpec((_W, o), lambda i: (i, 0)),
        out_shape=jax.ShapeDtypeStruct((n, o), jnp.float32),
    )(x, w)


def kernel(pos, shifts, cell, scalar_features, vector_features, params,
           edge_index, z):
    N = pos.shape[0]
    E = edge_index.shape[1]
    NSUB = (N + _W - 1) // _W
    N_pad = NSUB * _W
    NCHUNK = (E + _B - 1) // _B + NSUB
    E_pad = NCHUNK * _B

    src = edge_index[0]
    dst = edge_index[1]

    order = jnp.argsort(dst)
    dst_s = dst[order]
    src_s = src[order]
    bnd = jnp.searchsorted(
        dst_s, (_W * jnp.arange(NSUB + 1)).astype(dst_s.dtype),
        side='left').astype(jnp.int32)
    counts = bnd[1:] - bnd[:-1]
    off = bnd[:-1]
    nch = (jnp.maximum(counts, 1) + _B - 1) // _B
    cumch = jnp.cumsum(nch)
    chunk_ids = jnp.arange(NCHUNK, dtype=jnp.int32)
    boc = jnp.minimum(jnp.searchsorted(cumch, chunk_ids, side='right'),
                      NSUB - 1).astype(jnp.int32)
    chunk_start = (cumch - nch)[boc]
    first = (chunk_ids == chunk_start).astype(jnp.int32)

    padc = nch * _B
    pcum = jnp.cumsum(padc)
    p = jnp.arange(E_pad, dtype=jnp.int32)
    bp_raw = jnp.searchsorted(pcum, p, side='right')
    inb = bp_raw < NSUB
    bp = jnp.minimum(bp_raw, NSUB - 1).astype(jnp.int32)
    q = p - (pcum[bp] - padc[bp])
    valid = inb & (q < counts[bp])
    eidx = jnp.clip(off[bp] + jnp.minimum(q, jnp.maximum(counts[bp] - 1, 0)),
                    0, E - 1)
    src_pad = jnp.where(valid, src_s[eidx], 0).astype(jnp.int32)
    dst_pad = jnp.where(valid, dst_s[eidx], 0).astype(jnp.int32)
    dl = jnp.where(valid, dst_pad - _W * bp, -1).astype(jnp.int32)
    dl3 = dl.reshape(NCHUNK, 1, _B)

    ptab = jnp.pad(pos, ((0, 0), (0, 125)))
    ps = _sc_gather(ptab, src_pad)
    pd = _sc_gather(ptab, dst_pad)
    feat = pl.pallas_call(
        _geom_body,
        grid=(pl.cdiv(E_pad, _TE),),
        in_specs=[pl.BlockSpec((_TE, 128), lambda i: (i, 0)),
                  pl.BlockSpec((_TE, 128), lambda i: (i, 0))],
        out_specs=pl.BlockSpec((_TE, 64), lambda i: (i, 0)),
        out_shape=jax.ShapeDtypeStruct((E_pad, 64), jnp.float32),
    )(ps, pd)

    sf = jnp.nan_to_num(scalar_features)
    vf = jnp.nan_to_num(vector_features)
    x = jnp.concatenate([sf, vf.reshape(vf.shape[0], -1)], axis=-1)
    x = jnp.pad(x, ((0, N_pad - N), (0, 0)))

    h = _matmul(x, params["W_pre_0"])

    for i in range(_NLAYERS):
        h_src = _sc_gather(jnp.pad(h, ((0, 0), (0, 64))), src_pad)
        b1 = params[f"b1_{i}"].reshape(1, _C)
        agg = pl.pallas_call(
            _edge_agg_body,
            grid_spec=pltpu.PrefetchScalarGridSpec(
                num_scalar_prefetch=2,
                grid=(NCHUNK,),
                in_specs=[
                    pl.BlockSpec((_B, 64), lambda j, b, f: (j, 0)),
                    pl.BlockSpec((_B, 128), lambda j, b, f: (j, 0)),
                    pl.BlockSpec((1, 1, _B), lambda j, b, f: (j, 0, 0)),
                    pl.BlockSpec((_NB, _C), lambda j, b, f: (0, 0)),
                    pl.BlockSpec((1, _C), lambda j, b, f: (0, 0)),
                    pl.BlockSpec((_C, _C), lambda j, b, f: (0, 0)),
                ],
                out_specs=pl.BlockSpec((_W, _NSH * _C),
                                       lambda j, b, f: (b[j], 0)),
            ),
            out_shape=jax.ShapeDtypeStruct((N_pad, _NSH * _C), jnp.float32),
        )(boc, first, feat, h_src, dl3,
          params[f"W1_{i}"], b1, params[f"W2_{i}"])

        with_h = i + 1 < _NLAYERS
        d_in = x.shape[1]
        body = functools.partial(_node_out_body, with_h=with_h)
        in_specs = [
            pl.BlockSpec((_W, _NSH * _C), lambda t: (t, 0)),
            pl.BlockSpec((_W, d_in), lambda t: (t, 0)),
            pl.BlockSpec((_NSH * _C, _DIMH), lambda t: (0, 0)),
            pl.BlockSpec((d_in, _DIMH), lambda t: (0, 0)),
        ]
        operands = [agg, x, params[f"W_out_{i}"], params[f"W_self_{i}"]]
        if with_h:
            in_specs.append(pl.BlockSpec((_DIMH, _C), lambda t: (0, 0)))
            operands.append(params[f"W_pre_{i + 1}"])
            out_specs = [pl.BlockSpec((_W, _DIMH), lambda t: (t, 0)),
                         pl.BlockSpec((_W, _C), lambda t: (t, 0))]
            out_shape = [jax.ShapeDtypeStruct((N_pad, _DIMH), jnp.float32),
                         jax.ShapeDtypeStruct((N_pad, _C), jnp.float32)]
        else:
            out_specs = [pl.BlockSpec((_W, _DIMH), lambda t: (t, 0))]
            out_shape = [jax.ShapeDtypeStruct((N_pad, _DIMH), jnp.float32)]
        res = pl.pallas_call(
            body,
            grid=(N_pad // _W,),
            in_specs=in_specs,
            out_specs=out_specs,
            out_shape=out_shape,
        )(*operands)
        if with_h:
            x, h = res
        else:
            x = res[0]

    xo = x[:N]
    return (xo, xo[:, :_HS], xo[:, _HS:].reshape(N, _HV, 3))


# speedup vs baseline: 3.0680x; 3.0680x over previous
import functools

import jax
import jax.numpy as jnp
import numpy as np
from jax.experimental import pallas as pl
from jax.experimental.pallas import tpu as pltpu
from jax.experimental.pallas import tpu_sc as plsc

_NB = 32
_CUT = 5.0
_C = 64
_NSH = 9
_HS = 64
_HV = 32
_DIMH = _HS + 3 * _HV
_NLAYERS = 3

_W = 256
_B = 256
_TE = 1024


def _sc_gather(table, idx):
    m = idx.shape[0]
    n, d = table.shape
    win = 256
    mesh = plsc.VectorSubcoreMesh(core_axis_name="c", subcore_axis_name="s")

    @functools.partial(
        pl.kernel, mesh=mesh,
        out_type=jax.ShapeDtypeStruct((m, d), table.dtype))
    def k(x_hbm, i_hbm, o_hbm):
        def body(i_vmem, o_vmem):
            pltpu.sync_copy(x_hbm.at[i_vmem.at[0]], o_vmem)

        pltpu.emit_pipeline(
            body,
            grid=(m // win,),
            in_specs=[pl.BlockSpec((1, win), lambda i: (0, i))],
            out_specs=[pl.BlockSpec((win, d), lambda i: (i, 0))],
            core_axis_name=("c", "s"),
            dimension_semantics=(pltpu.PARALLEL,),
        )(i_hbm, o_hbm)

    return k(table, idx.reshape(1, m))


def _geom_body(ps_ref, pd_ref, feat_ref):
    ps = ps_ref[:, 0:3]
    pd = pd_ref[:, 0:3]
    v = pd - ps
    r = jnp.sqrt(jnp.sum(v * v, axis=1, keepdims=True) + 1e-16)
    d = v / jnp.maximum(r, 1e-8)
    x = d[:, 0:1]
    y = d[:, 1:2]
    z = d[:, 2:3]
    c1 = np.float32(np.sqrt(3.0))
    c2 = np.float32(np.sqrt(15.0))
    c3 = np.float32(np.sqrt(5.0) / 2.0)
    sh = jnp.concatenate([
        jnp.ones_like(x), c1 * y, c1 * z, c1 * x,
        c2 * x * y, c2 * y * z, c3 * (3.0 * z * z - 1.0), c2 * x * z,
        (c2 / 2.0) * (x * x - y * y)
    ], axis=1)
    u = (r / _CUT) ** 2
    us = jnp.minimum(u, 0.99)
    env = jnp.where(r < _CUT, jnp.exp(1.0 - 1.0 / (1.0 - us)), 0.0)
    step = np.float32(_CUT / (_NB - 1))
    centers = (jax.lax.broadcasted_iota(jnp.int32, (1, _NB), 1)
               .astype(jnp.float32) * step)
    width = _CUT / _NB
    g = jnp.exp(-((r - centers) ** 2) / (2.0 * width * width))
    ea = g * env
    shg = sh * env
    pad = jnp.zeros((ps.shape[0], 64 - _NB - _NSH), dtype=jnp.float32)
    feat_ref[...] = jnp.concatenate([ea, shg, pad], axis=1)


def _mm_body(x_ref, w_ref, o_ref):
    o_ref[...] = jax.lax.dot_general(
        x_ref[...], w_ref[...], (((1,), (0,)), ((), ())),
        preferred_element_type=jnp.float32)


def _edge_agg_body(boc_ref, first_ref, feat_ref, hsrc_ref, dl_ref,
                   w1_ref, b1_ref, w2_ref, out_ref):
    j = pl.program_id(0)
    ea = feat_ref[:, 0:_NB]
    shg = feat_ref[:, _NB:_NB + _NSH]
    t = jax.lax.dot_general(ea, w1_ref[...], (((1,), (0,)), ((), ())),
                            preferred_element_type=jnp.float32)
    t = jnp.maximum(t + b1_ref[...], 0.0)
    R = jax.lax.dot_general(t, w2_ref[...], (((1,), (0,)), ((), ())),
                            preferred_element_type=jnp.float32)
    m = hsrc_ref[:, 0:_C] * R
    msh = jnp.concatenate(
        [m * shg[:, k:k + 1] for k in range(_NSH)], axis=1
    ).astype(jnp.bfloat16)
    dl = dl_ref[0, 0, :]
    oh = (jax.lax.broadcasted_iota(jnp.int32, (_W, _B), 0)
          == dl[None, :]).astype(jnp.bfloat16)
    contrib = jax.lax.dot_general(oh, msh, (((1,), (0,)), ((), ())),
                                  preferred_element_type=jnp.float32)
    first = first_ref[j]

    @pl.when(first == 1)
    def _():
        out_ref[...] = contrib

    @pl.when(first == 0)
    def _():
        out_ref[...] += contrib


def _node_out_body(agg_ref, x_ref, wout_ref, wself_ref, *rest, with_h):
    if with_h:
        wpre_ref, out_ref, h_ref = rest
    else:
        (out_ref,) = rest
    out = jax.lax.dot_general(agg_ref[...], wout_ref[...],
                              (((1,), (0,)), ((), ())),
                              preferred_element_type=jnp.float32)
    out += jax.lax.dot_general(x_ref[...], wself_ref[...],
                               (((1,), (0,)), ((), ())),
                               preferred_element_type=jnp.float32)
    out_ref[...] = out
    if with_h:
        h_ref[...] = jax.lax.dot_general(out, wpre_ref[...],
                                         (((1,), (0,)), ((), ())),
                                         preferred_element_type=jnp.float32)


def _matmul(x, w):
    n, d = x.shape
    _, o = w.shape
    return pl.pallas_call(
        _mm_body,
        grid=(n // _W,),
        in_specs=[pl.BlockSpec((_W, d), lambda i: (i, 0)),
                  pl.BlockSpec((d, o), lambda i: (0, 0))],
        out_specs=pl.BlockSpec((_W, o), lambda i: (i, 0)),
        out_shape=jax.ShapeDtypeStruct((n, o), jnp.float32),
    )(x, w)


def kernel(pos, shifts, cell, scalar_features, vector_features, params,
           edge_index, z):
    N = pos.shape[0]
    E = edge_index.shape[1]
    NSUB = (N + _W - 1) // _W
    N_pad = NSUB * _W
    NCHUNK = (E + _B - 1) // _B + NSUB
    E_pad = NCHUNK * _B

    src = edge_index[0]
    dst = edge_index[1]

    w_shift = int(np.log2(_W))
    assert (1 << w_shift) == _W
    NSUBP = ((NSUB + 127) // 128) * 128
    order = jnp.argsort(dst)
    dst_s = dst[order]
    src_s = src[order]
    bnd = jnp.searchsorted(
        dst_s, (_W * jnp.arange(NSUB + 1)).astype(dst_s.dtype),
        side='left').astype(jnp.int32)
    counts = bnd[1:] - bnd[:-1]
    off = bnd[:-1]
    nch = (jnp.maximum(counts, 1) + _B - 1) // _B
    cumch = jnp.cumsum(nch)
    chunk_ids = jnp.arange(NCHUNK, dtype=jnp.int32)
    boc = jnp.minimum(jnp.searchsorted(cumch, chunk_ids, side='right'),
                      NSUB - 1).astype(jnp.int32)
    chunk_start = (cumch - nch)[boc]
    first = (chunk_ids == chunk_start).astype(jnp.int32)
    pad_off = (cumch - nch) * _B

    delta = (pad_off - off).astype(jnp.float32)
    delta_t = jnp.pad(delta, (0, NSUBP - NSUB)).reshape(1, NSUBP)
    SUBR = 8
    G2 = SUBR * _B
    NCH2 = (E + G2 - 1) // G2
    E2 = NCH2 * G2
    dst_s3 = jnp.pad(dst_s, (0, E2 - E), constant_values=N_pad
                     ).reshape(NCH2, SUBR, _B)

    def _place_body(d_ref, t_ref, pos_ref):
        g = pl.program_id(0)
        d = d_ref[0]
        t = t_ref[...]
        b = jax.lax.shift_right_logical(d, w_shift)
        rows = []
        for k in range(SUBR):
            bk = b[k]
            O32 = (bk[:, None] == jax.lax.broadcasted_iota(
                jnp.int32, (_B, NSUBP), 1)).astype(jnp.float32)
            dlt = jnp.sum(O32 * t, axis=1)
            e_glob = (g * G2 + k * _B
                      + jax.lax.broadcasted_iota(jnp.int32, (_B,), 0))
            pk = e_glob.astype(jnp.float32) + dlt
            rows.append(jnp.where(bk < NSUB, pk, -1.0))
        pos_ref[0] = jnp.stack(rows, 0).astype(jnp.int32)

    position = pl.pallas_call(
        _place_body,
        grid=(NCH2,),
        in_specs=[pl.BlockSpec((1, SUBR, _B), lambda g: (g, 0, 0)),
                  pl.BlockSpec((1, NSUBP), lambda g: (0, 0))],
        out_specs=pl.BlockSpec((1, SUBR, _B), lambda g: (g, 0, 0)),
        out_shape=jax.ShapeDtypeStruct((NCH2, SUBR, _B), jnp.int32),
    )(dst_s3, delta_t).reshape(E2)[:E]

    init = jnp.concatenate(
        [jnp.zeros((E_pad, 1), jnp.int32),
         jnp.full((E_pad, 1), -1, jnp.int32)], axis=1)
    pair = init.at[position].set(
        jnp.stack([src_s, dst_s], axis=1), mode='drop')
    src_pad = pair[:, 0]
    dstp = pair[:, 1]
    dst_pad = jnp.maximum(dstp, 0)
    dl = jnp.where(dstp >= 0, dstp & (_W - 1), -1).astype(jnp.int32)
    dl3 = dl.reshape(NCHUNK, 1, _B)

    ptab = jnp.pad(pos, ((0, 0), (0, 125)))
    ps = _sc_gather(ptab, src_pad)
    pd = _sc_gather(ptab, dst_pad)
    feat = pl.pallas_call(
        _geom_body,
        grid=(pl.cdiv(E_pad, _TE),),
        in_specs=[pl.BlockSpec((_TE, 128), lambda i: (i, 0)),
                  pl.BlockSpec((_TE, 128), lambda i: (i, 0))],
        out_specs=pl.BlockSpec((_TE, 64), lambda i: (i, 0)),
        out_shape=jax.ShapeDtypeStruct((E_pad, 64), jnp.float32),
    )(ps, pd)

    sf = jnp.nan_to_num(scalar_features)
    vf = jnp.nan_to_num(vector_features)
    x = jnp.concatenate([sf, vf.reshape(vf.shape[0], -1)], axis=-1)
    x = jnp.pad(x, ((0, N_pad - N), (0, 0)))

    h = _matmul(x, params["W_pre_0"])

    for i in range(_NLAYERS):
        h_src = _sc_gather(jnp.pad(h, ((0, 0), (0, 64))), src_pad)
        b1 = params[f"b1_{i}"].reshape(1, _C)
        agg = pl.pallas_call(
            _edge_agg_body,
            grid_spec=pltpu.PrefetchScalarGridSpec(
                num_scalar_prefetch=2,
                grid=(NCHUNK,),
                in_specs=[
                    pl.BlockSpec((_B, 64), lambda j, b, f: (j, 0)),
                    pl.BlockSpec((_B, 128), lambda j, b, f: (j, 0)),
                    pl.BlockSpec((1, 1, _B), lambda j, b, f: (j, 0, 0)),
                    pl.BlockSpec((_NB, _C), lambda j, b, f: (0, 0)),
                    pl.BlockSpec((1, _C), lambda j, b, f: (0, 0)),
                    pl.BlockSpec((_C, _C), lambda j, b, f: (0, 0)),
                ],
                out_specs=pl.BlockSpec((_W, _NSH * _C),
                                       lambda j, b, f: (b[j], 0)),
            ),
            out_shape=jax.ShapeDtypeStruct((N_pad, _NSH * _C), jnp.float32),
        )(boc, first, feat, h_src, dl3,
          params[f"W1_{i}"], b1, params[f"W2_{i}"])

        with_h = i + 1 < _NLAYERS
        d_in = x.shape[1]
        body = functools.partial(_node_out_body, with_h=with_h)
        in_specs = [
            pl.BlockSpec((_W, _NSH * _C), lambda t: (t, 0)),
            pl.BlockSpec((_W, d_in), lambda t: (t, 0)),
            pl.BlockSpec((_NSH * _C, _DIMH), lambda t: (0, 0)),
            pl.BlockSpec((d_in, _DIMH), lambda t: (0, 0)),
        ]
        operands = [agg, x, params[f"W_out_{i}"], params[f"W_self_{i}"]]
        if with_h:
            in_specs.append(pl.BlockSpec((_DIMH, _C), lambda t: (0, 0)))
            operands.append(params[f"W_pre_{i + 1}"])
            out_specs = [pl.BlockSpec((_W, _DIMH), lambda t: (t, 0)),
                         pl.BlockSpec((_W, _C), lambda t: (t, 0))]
            out_shape = [jax.ShapeDtypeStruct((N_pad, _DIMH), jnp.float32),
                         jax.ShapeDtypeStruct((N_pad, _C), jnp.float32)]
        else:
            out_specs = [pl.BlockSpec((_W, _DIMH), lambda t: (t, 0))]
            out_shape = [jax.ShapeDtypeStruct((N_pad, _DIMH), jnp.float32)]
        res = pl.pallas_call(
            body,
            grid=(N_pad // _W,),
            in_specs=in_specs,
            out_specs=out_specs,
            out_shape=out_shape,
        )(*operands)
        if with_h:
            x, h = res
        else:
            x = res[0]

    xo = x[:N]
    return (xo, xo[:, :_HS], xo[:, _HS:].reshape(N, _HV, 3))


# speedup vs baseline: 3.2568x; 1.0615x over previous
import functools

import jax
import jax.numpy as jnp
import numpy as np
from jax.experimental import pallas as pl
from jax.experimental.pallas import tpu as pltpu
from jax.experimental.pallas import tpu_sc as plsc

_NB = 32
_CUT = 5.0
_C = 64
_NSH = 9
_HS = 64
_HV = 32
_DIMH = _HS + 3 * _HV
_NLAYERS = 3

_W = 256
_B = 256
_TE = 1024


def _sc_gather(table, idx):
    m = idx.shape[0]
    n, d = table.shape
    win = 256
    mesh = plsc.VectorSubcoreMesh(core_axis_name="c", subcore_axis_name="s")

    @functools.partial(
        pl.kernel, mesh=mesh,
        out_type=jax.ShapeDtypeStruct((m, d), table.dtype))
    def k(x_hbm, i_hbm, o_hbm):
        def body(i_vmem, o_vmem):
            pltpu.sync_copy(x_hbm.at[i_vmem.at[0]], o_vmem)

        pltpu.emit_pipeline(
            body,
            grid=(m // win,),
            in_specs=[pl.BlockSpec((1, win), lambda i: (0, i))],
            out_specs=[pl.BlockSpec((win, d), lambda i: (i, 0))],
            core_axis_name=("c", "s"),
            dimension_semantics=(pltpu.PARALLEL,),
        )(i_hbm, o_hbm)

    return k(table, idx.reshape(1, m))


def _geom_math(ps, pd):
    v = pd[:, 0:3] - ps[:, 0:3]
    r = jnp.sqrt(jnp.sum(v * v, axis=1, keepdims=True) + 1e-16)
    d = v / jnp.maximum(r, 1e-8)
    x = d[:, 0:1]
    y = d[:, 1:2]
    z = d[:, 2:3]
    c1 = np.float32(np.sqrt(3.0))
    c2 = np.float32(np.sqrt(15.0))
    c3 = np.float32(np.sqrt(5.0) / 2.0)
    sh = jnp.concatenate([
        jnp.ones_like(x), c1 * y, c1 * z, c1 * x,
        c2 * x * y, c2 * y * z, c3 * (3.0 * z * z - 1.0), c2 * x * z,
        (c2 / 2.0) * (x * x - y * y)
    ], axis=1)
    u = (r / _CUT) ** 2
    us = jnp.minimum(u, 0.99)
    env = jnp.where(r < _CUT, jnp.exp(1.0 - 1.0 / (1.0 - us)), 0.0)
    step = np.float32(_CUT / (_NB - 1))
    centers = (jax.lax.broadcasted_iota(jnp.int32, (1, _NB), 1)
               .astype(jnp.float32) * step)
    width = _CUT / _NB
    g = jnp.exp(-((r - centers) ** 2) / (2.0 * width * width))
    return g * env, sh * env


def _msg_agg(ea, shg, hsrc64, dl, w1_ref, b1_ref, w2_ref):
    t = jax.lax.dot_general(ea, w1_ref[...], (((1,), (0,)), ((), ())),
                            preferred_element_type=jnp.float32)
    t = jnp.maximum(t + b1_ref[...], 0.0)
    R = jax.lax.dot_general(t, w2_ref[...], (((1,), (0,)), ((), ())),
                            preferred_element_type=jnp.float32)
    m = hsrc64 * R
    msh = jnp.concatenate(
        [m * shg[:, k:k + 1] for k in range(_NSH)], axis=1
    ).astype(jnp.bfloat16)
    oh = (jax.lax.broadcasted_iota(jnp.int32, (_W, _B), 0)
          == dl[None, :]).astype(jnp.bfloat16)
    return jax.lax.dot_general(oh, msh, (((1,), (0,)), ((), ())),
                               preferred_element_type=jnp.float32)


def _edge_geom_agg_body(boc_ref, first_ref, g_ref, posb_ref, dl_ref,
                        w1_ref, b1_ref, w2_ref, agg_ref, feat_ref):
    j = pl.program_id(0)
    dl = dl_ref[0, 0, :]
    ohbw = (dl[:, None] == jax.lax.broadcasted_iota(
        jnp.int32, (_B, _W), 1)).astype(jnp.float32)
    pd = jax.lax.dot_general(ohbw, posb_ref[...], (((1,), (0,)), ((), ())),
                             precision=jax.lax.Precision.HIGHEST,
                             preferred_element_type=jnp.float32)
    ps = g_ref[:, 64:68]
    ea, shg = _geom_math(ps, pd)
    pad = jnp.zeros((ea.shape[0], 64 - _NB - _NSH), dtype=jnp.float32)
    feat_ref[...] = jnp.concatenate([ea, shg, pad], axis=1)
    contrib = _msg_agg(ea, shg, g_ref[:, 0:_C], dl, w1_ref, b1_ref, w2_ref)
    first = first_ref[j]

    @pl.when(first == 1)
    def _():
        agg_ref[...] = contrib

    @pl.when(first == 0)
    def _():
        agg_ref[...] += contrib


def _mm_body(x_ref, w_ref, o_ref):
    o_ref[...] = jax.lax.dot_general(
        x_ref[...], w_ref[...], (((1,), (0,)), ((), ())),
        preferred_element_type=jnp.float32)


def _edge_agg_body(boc_ref, first_ref, feat_ref, hsrc_ref, dl_ref,
                   w1_ref, b1_ref, w2_ref, out_ref):
    j = pl.program_id(0)
    ea = feat_ref[:, 0:_NB]
    shg = feat_ref[:, _NB:_NB + _NSH]
    dl = dl_ref[0, 0, :]
    contrib = _msg_agg(ea, shg, hsrc_ref[:, 0:_C], dl,
                       w1_ref, b1_ref, w2_ref)
    first = first_ref[j]

    @pl.when(first == 1)
    def _():
        out_ref[...] = contrib

    @pl.when(first == 0)
    def _():
        out_ref[...] += contrib


def _node_out_body(agg_ref, x_ref, wout_ref, wself_ref, *rest, with_h):
    if with_h:
        wpre_ref, out_ref, h_ref = rest
    else:
        (out_ref,) = rest
    out = jax.lax.dot_general(agg_ref[...], wout_ref[...],
                              (((1,), (0,)), ((), ())),
                              preferred_element_type=jnp.float32)
    out += jax.lax.dot_general(x_ref[...], wself_ref[...],
                               (((1,), (0,)), ((), ())),
                               preferred_element_type=jnp.float32)
    out_ref[...] = out
    if with_h:
        h_ref[...] = jax.lax.dot_general(out, wpre_ref[...],
                                         (((1,), (0,)), ((), ())),
                                         preferred_element_type=jnp.float32)


def _matmul(x, w):
    n, d = x.shape
    _, o = w.shape
    return pl.pallas_call(
        _mm_body,
        grid=(n // _W,),
        in_specs=[pl.BlockSpec((_W, d), lambda i: (i, 0)),
                  pl.BlockSpec((d, o), lambda i: (0, 0))],
        out_specs=pl.BlockSpec((_W, o), lambda i: (i, 0)),
        out_shape=jax.ShapeDtypeStruct((n, o), jnp.float32),
    )(x, w)


def kernel(pos, shifts, cell, scalar_features, vector_features, params,
           edge_index, z):
    N = pos.shape[0]
    E = edge_index.shape[1]
    NSUB = (N + _W - 1) // _W
    N_pad = NSUB * _W
    NCHUNK = (E + _B - 1) // _B + NSUB
    E_pad = NCHUNK * _B

    src = edge_index[0]
    dst = edge_index[1]

    w_shift = int(np.log2(_W))
    assert (1 << w_shift) == _W
    NSUBP = ((NSUB + 127) // 128) * 128
    order = jnp.argsort(dst)
    dst_s = dst[order]
    src_s = src[order]
    bnd = jnp.searchsorted(
        dst_s, (_W * jnp.arange(NSUB + 1)).astype(dst_s.dtype),
        side='left').astype(jnp.int32)
    counts = bnd[1:] - bnd[:-1]
    off = bnd[:-1]
    nch = (jnp.maximum(counts, 1) + _B - 1) // _B
    cumch = jnp.cumsum(nch)
    chunk_ids = jnp.arange(NCHUNK, dtype=jnp.int32)
    boc = jnp.minimum(jnp.searchsorted(cumch, chunk_ids, side='right'),
                      NSUB - 1).astype(jnp.int32)
    chunk_start = (cumch - nch)[boc]
    first = (chunk_ids == chunk_start).astype(jnp.int32)
    pad_off = (cumch - nch) * _B

    delta = (pad_off - off).astype(jnp.float32)
    delta_t = jnp.pad(delta, (0, NSUBP - NSUB)).reshape(1, NSUBP)
    SUBR = 8
    G2 = SUBR * _B
    NCH2 = (E + G2 - 1) // G2
    E2 = NCH2 * G2
    dst_s3 = jnp.pad(dst_s, (0, E2 - E), constant_values=N_pad
                     ).reshape(NCH2, SUBR, _B)

    def _place_body(d_ref, t_ref, pos_ref):
        g = pl.program_id(0)
        d = d_ref[0]
        t = t_ref[...]
        b = jax.lax.shift_right_logical(d, w_shift)
        rows = []
        for k in range(SUBR):
            bk = b[k]
            O32 = (bk[:, None] == jax.lax.broadcasted_iota(
                jnp.int32, (_B, NSUBP), 1)).astype(jnp.float32)
            dlt = jnp.sum(O32 * t, axis=1)
            e_glob = (g * G2 + k * _B
                      + jax.lax.broadcasted_iota(jnp.int32, (_B,), 0))
            pk = e_glob.astype(jnp.float32) + dlt
            rows.append(jnp.where(bk < NSUB, pk, -1.0))
        pos_ref[0] = jnp.stack(rows, 0).astype(jnp.int32)

    position = pl.pallas_call(
        _place_body,
        grid=(NCH2,),
        in_specs=[pl.BlockSpec((1, SUBR, _B), lambda g: (g, 0, 0)),
                  pl.BlockSpec((1, NSUBP), lambda g: (0, 0))],
        out_specs=pl.BlockSpec((1, SUBR, _B), lambda g: (g, 0, 0)),
        out_shape=jax.ShapeDtypeStruct((NCH2, SUBR, _B), jnp.int32),
    )(dst_s3, delta_t).reshape(E2)[:E]

    init = jnp.concatenate(
        [jnp.zeros((E_pad, 1), jnp.int32),
         jnp.full((E_pad, 1), -1, jnp.int32)], axis=1)
    pair = init.at[position].set(
        jnp.stack([src_s, dst_s], axis=1), mode='drop')
    src_pad = pair[:, 0]
    dstp = pair[:, 1]
    dl = jnp.where(dstp >= 0, dstp & (_W - 1), -1).astype(jnp.int32)
    dl3 = dl.reshape(NCHUNK, 1, _B)

    sf = jnp.nan_to_num(scalar_features)
    vf = jnp.nan_to_num(vector_features)
    x = jnp.concatenate([sf, vf.reshape(vf.shape[0], -1)], axis=-1)
    x = jnp.pad(x, ((0, N_pad - N), (0, 0)))

    pos4 = jnp.pad(pos, ((0, N_pad - N), (0, 1)))
    h0 = _matmul(x, params["W_pre_0"])
    table0 = jnp.concatenate(
        [h0, pos4, jnp.zeros((N_pad, 60), jnp.float32)], axis=1)

    feat = None
    for i in range(_NLAYERS):
        if i == 0:
            g = _sc_gather(table0, src_pad)
        else:
            g = _sc_gather(jnp.pad(h, ((0, 0), (0, 64))), src_pad)
        b1 = params[f"b1_{i}"].reshape(1, _C)
        common_specs = [
            pl.BlockSpec((1, 1, _B), lambda j, b, f: (j, 0, 0)),
            pl.BlockSpec((_NB, _C), lambda j, b, f: (0, 0)),
            pl.BlockSpec((1, _C), lambda j, b, f: (0, 0)),
            pl.BlockSpec((_C, _C), lambda j, b, f: (0, 0)),
        ]
        agg_spec = pl.BlockSpec((_W, _NSH * _C), lambda j, b, f: (b[j], 0))
        wargs = (params[f"W1_{i}"], b1, params[f"W2_{i}"])
        if i == 0:
            agg, feat = pl.pallas_call(
                _edge_geom_agg_body,
                grid_spec=pltpu.PrefetchScalarGridSpec(
                    num_scalar_prefetch=2,
                    grid=(NCHUNK,),
                    in_specs=[
                        pl.BlockSpec((_B, 128), lambda j, b, f: (j, 0)),
                        pl.BlockSpec((_W, 4), lambda j, b, f: (b[j], 0)),
                    ] + common_specs,
                    out_specs=[agg_spec,
                               pl.BlockSpec((_B, 64), lambda j, b, f: (j, 0))],
                ),
                out_shape=[
                    jax.ShapeDtypeStruct((N_pad, _NSH * _C), jnp.float32),
                    jax.ShapeDtypeStruct((E_pad, 64), jnp.float32)],
            )(boc, first, g, pos4, dl3, *wargs)
        else:
            agg = pl.pallas_call(
                _edge_agg_body,
                grid_spec=pltpu.PrefetchScalarGridSpec(
                    num_scalar_prefetch=2,
                    grid=(NCHUNK,),
                    in_specs=[
                        pl.BlockSpec((_B, 64), lambda j, b, f: (j, 0)),
                        pl.BlockSpec((_B, 128), lambda j, b, f: (j, 0)),
                    ] + common_specs,
                    out_specs=agg_spec,
                ),
                out_shape=jax.ShapeDtypeStruct((N_pad, _NSH * _C),
                                               jnp.float32),
            )(boc, first, feat, g, dl3, *wargs)

        with_h = i + 1 < _NLAYERS
        d_in = x.shape[1]
        body = functools.partial(_node_out_body, with_h=with_h)
        in_specs = [
            pl.BlockSpec((_W, _NSH * _C), lambda t: (t, 0)),
            pl.BlockSpec((_W, d_in), lambda t: (t, 0)),
            pl.BlockSpec((_NSH * _C, _DIMH), lambda t: (0, 0)),
            pl.BlockSpec((d_in, _DIMH), lambda t: (0, 0)),
        ]
        operands = [agg, x, params[f"W_out_{i}"], params[f"W_self_{i}"]]
        if with_h:
            in_specs.append(pl.BlockSpec((_DIMH, _C), lambda t: (0, 0)))
            operands.append(params[f"W_pre_{i + 1}"])
            out_specs = [pl.BlockSpec((_W, _DIMH), lambda t: (t, 0)),
                         pl.BlockSpec((_W, _C), lambda t: (t, 0))]
            out_shape = [jax.ShapeDtypeStruct((N_pad, _DIMH), jnp.float32),
                         jax.ShapeDtypeStruct((N_pad, _C), jnp.float32)]
        else:
            out_specs = [pl.BlockSpec((_W, _DIMH), lambda t: (t, 0))]
            out_shape = [jax.ShapeDtypeStruct((N_pad, _DIMH), jnp.float32)]
        res = pl.pallas_call(
            body,
            grid=(N_pad // _W,),
            in_specs=in_specs,
            out_specs=out_specs,
            out_shape=out_shape,
        )(*operands)
        if with_h:
            x, h = res
        else:
            x = res[0]

    xo = x[:N]
    return (xo, xo[:, :_HS], xo[:, _HS:].reshape(N, _HV, 3))


# speedup vs baseline: 3.3989x; 1.0436x over previous
import functools

import jax
import jax.numpy as jnp
import numpy as np
from jax.experimental import pallas as pl
from jax.experimental.pallas import tpu as pltpu
from jax.experimental.pallas import tpu_sc as plsc

_NB = 32
_CUT = 5.0
_C = 64
_NSH = 9
_HS = 64
_HV = 32
_DIMH = _HS + 3 * _HV
_NLAYERS = 3

_W = 512
_B = 512
_TE = 1024


def _sc_gather(table, idx):
    m = idx.shape[0]
    n, d = table.shape
    win = 256
    mesh = plsc.VectorSubcoreMesh(core_axis_name="c", subcore_axis_name="s")

    @functools.partial(
        pl.kernel, mesh=mesh,
        out_type=jax.ShapeDtypeStruct((m, d), table.dtype))
    def k(x_hbm, i_hbm, o_hbm):
        def body(i_vmem, o_vmem):
            pltpu.sync_copy(x_hbm.at[i_vmem.at[0]], o_vmem)

        pltpu.emit_pipeline(
            body,
            grid=(m // win,),
            in_specs=[pl.BlockSpec((1, win), lambda i: (0, i))],
            out_specs=[pl.BlockSpec((win, d), lambda i: (i, 0))],
            core_axis_name=("c", "s"),
            dimension_semantics=(pltpu.PARALLEL,),
        )(i_hbm, o_hbm)

    return k(table, idx.reshape(1, m))


def _geom_math(ps, pd):
    v = pd[:, 0:3] - ps[:, 0:3]
    r = jnp.sqrt(jnp.sum(v * v, axis=1, keepdims=True) + 1e-16)
    d = v / jnp.maximum(r, 1e-8)
    x = d[:, 0:1]
    y = d[:, 1:2]
    z = d[:, 2:3]
    c1 = np.float32(np.sqrt(3.0))
    c2 = np.float32(np.sqrt(15.0))
    c3 = np.float32(np.sqrt(5.0) / 2.0)
    sh = jnp.concatenate([
        jnp.ones_like(x), c1 * y, c1 * z, c1 * x,
        c2 * x * y, c2 * y * z, c3 * (3.0 * z * z - 1.0), c2 * x * z,
        (c2 / 2.0) * (x * x - y * y)
    ], axis=1)
    u = (r / _CUT) ** 2
    us = jnp.minimum(u, 0.99)
    env = jnp.where(r < _CUT, jnp.exp(1.0 - 1.0 / (1.0 - us)), 0.0)
    step = np.float32(_CUT / (_NB - 1))
    centers = (jax.lax.broadcasted_iota(jnp.int32, (1, _NB), 1)
               .astype(jnp.float32) * step)
    width = _CUT / _NB
    g = jnp.exp(-((r - centers) ** 2) / (2.0 * width * width))
    return g * env, sh * env


def _msg_agg(ea, shg, hsrc64, dl, w1_ref, b1_ref, w2_ref):
    t = jax.lax.dot_general(ea.astype(jnp.bfloat16), w1_ref[...],
                            (((1,), (0,)), ((), ())),
                            preferred_element_type=jnp.float32)
    t = jnp.maximum(t + b1_ref[...], 0.0)
    R = jax.lax.dot_general(t.astype(jnp.bfloat16), w2_ref[...],
                            (((1,), (0,)), ((), ())),
                            preferred_element_type=jnp.float32)
    m = hsrc64 * R
    msh = jnp.concatenate(
        [m * shg[:, k:k + 1] for k in range(_NSH)], axis=1
    ).astype(jnp.bfloat16)
    oh = (jax.lax.broadcasted_iota(jnp.int32, (_W, _B), 0)
          == dl[None, :]).astype(jnp.bfloat16)
    return jax.lax.dot_general(oh, msh, (((1,), (0,)), ((), ())),
                               preferred_element_type=jnp.float32)


def _edge_geom_agg_body(boc_ref, first_ref, g_ref, posb_ref, dl_ref,
                        w1_ref, b1_ref, w2_ref, agg_ref, feat_ref):
    j = pl.program_id(0)
    dl = dl_ref[0, 0, :]
    ohbw = (dl[:, None] == jax.lax.broadcasted_iota(
        jnp.int32, (_B, _W), 1)).astype(jnp.float32)
    pd = jax.lax.dot_general(ohbw, posb_ref[...], (((1,), (0,)), ((), ())),
                             precision=jax.lax.Precision.HIGHEST,
                             preferred_element_type=jnp.float32)
    ps = g_ref[:, 64:68]
    ea, shg = _geom_math(ps, pd)
    pad = jnp.zeros((ea.shape[0], 64 - _NB - _NSH), dtype=jnp.float32)
    feat_ref[...] = jnp.concatenate([ea, shg, pad], axis=1)
    contrib = _msg_agg(ea, shg, g_ref[:, 0:_C], dl, w1_ref, b1_ref, w2_ref)
    first = first_ref[j]

    @pl.when(first == 1)
    def _():
        agg_ref[...] = contrib

    @pl.when(first == 0)
    def _():
        agg_ref[...] += contrib


def _mm_body(x_ref, w_ref, o_ref):
    o_ref[...] = jax.lax.dot_general(
        x_ref[...], w_ref[...], (((1,), (0,)), ((), ())),
        preferred_element_type=jnp.float32)


def _edge_agg_body(boc_ref, first_ref, feat_ref, hsrc_ref, dl_ref,
                   w1_ref, b1_ref, w2_ref, out_ref):
    j = pl.program_id(0)
    ea = feat_ref[:, 0:_NB]
    shg = feat_ref[:, _NB:_NB + _NSH]
    dl = dl_ref[0, 0, :]
    contrib = _msg_agg(ea, shg, hsrc_ref[:, 0:_C], dl,
                       w1_ref, b1_ref, w2_ref)
    first = first_ref[j]

    @pl.when(first == 1)
    def _():
        out_ref[...] = contrib

    @pl.when(first == 0)
    def _():
        out_ref[...] += contrib


def _node_out_body(agg_ref, x_ref, wout_ref, wself_ref, *rest, with_h):
    if with_h:
        wpre_ref, out_ref, h_ref = rest
    else:
        (out_ref,) = rest
    out = jax.lax.dot_general(agg_ref[...], wout_ref[...],
                              (((1,), (0,)), ((), ())),
                              preferred_element_type=jnp.float32)
    out += jax.lax.dot_general(x_ref[...], wself_ref[...],
                               (((1,), (0,)), ((), ())),
                               preferred_element_type=jnp.float32)
    out_ref[...] = out
    if with_h:
        h_ref[...] = jax.lax.dot_general(out, wpre_ref[...],
                                         (((1,), (0,)), ((), ())),
                                         preferred_element_type=jnp.float32)


def _matmul(x, w):
    n, d = x.shape
    _, o = w.shape
    return pl.pallas_call(
        _mm_body,
        grid=(n // _W,),
        in_specs=[pl.BlockSpec((_W, d), lambda i: (i, 0)),
                  pl.BlockSpec((d, o), lambda i: (0, 0))],
        out_specs=pl.BlockSpec((_W, o), lambda i: (i, 0)),
        out_shape=jax.ShapeDtypeStruct((n, o), jnp.float32),
    )(x, w)


def kernel(pos, shifts, cell, scalar_features, vector_features, params,
           edge_index, z):
    N = pos.shape[0]
    E = edge_index.shape[1]
    NSUB = (N + _W - 1) // _W
    N_pad = NSUB * _W
    NCHUNK = (E + _B - 1) // _B + NSUB
    E_pad = NCHUNK * _B

    src = edge_index[0]
    dst = edge_index[1]

    w_shift = int(np.log2(_W))
    assert (1 << w_shift) == _W
    NSUBP = ((NSUB + 127) // 128) * 128
    order = jnp.argsort(dst)
    dst_s = dst[order]
    src_s = src[order]
    bnd = jnp.searchsorted(
        dst_s, (_W * jnp.arange(NSUB + 1)).astype(dst_s.dtype),
        side='left').astype(jnp.int32)
    counts = bnd[1:] - bnd[:-1]
    off = bnd[:-1]
    nch = (jnp.maximum(counts, 1) + _B - 1) // _B
    cumch = jnp.cumsum(nch)
    chunk_ids = jnp.arange(NCHUNK, dtype=jnp.int32)
    boc = jnp.minimum(jnp.searchsorted(cumch, chunk_ids, side='right'),
                      NSUB - 1).astype(jnp.int32)
    chunk_start = (cumch - nch)[boc]
    first = (chunk_ids == chunk_start).astype(jnp.int32)
    pad_off = (cumch - nch) * _B

    delta = (pad_off - off).astype(jnp.float32)
    delta_t = jnp.pad(delta, (0, NSUBP - NSUB)).reshape(1, NSUBP)
    SUBR = 8
    G2 = SUBR * _B
    NCH2 = (E + G2 - 1) // G2
    E2 = NCH2 * G2
    dst_s3 = jnp.pad(dst_s, (0, E2 - E), constant_values=N_pad
                     ).reshape(NCH2, SUBR, _B)

    def _place_body(d_ref, t_ref, pos_ref):
        g = pl.program_id(0)
        d = d_ref[0]
        t = t_ref[...]
        b = jax.lax.shift_right_logical(d, w_shift)
        rows = []
        for k in range(SUBR):
            bk = b[k]
            O32 = (bk[:, None] == jax.lax.broadcasted_iota(
                jnp.int32, (_B, NSUBP), 1)).astype(jnp.float32)
            dlt = jnp.sum(O32 * t, axis=1)
            e_glob = (g * G2 + k * _B
                      + jax.lax.broadcasted_iota(jnp.int32, (_B,), 0))
            pk = e_glob.astype(jnp.float32) + dlt
            rows.append(jnp.where(bk < NSUB, pk, -1.0))
        pos_ref[0] = jnp.stack(rows, 0).astype(jnp.int32)

    position = pl.pallas_call(
        _place_body,
        grid=(NCH2,),
        in_specs=[pl.BlockSpec((1, SUBR, _B), lambda g: (g, 0, 0)),
                  pl.BlockSpec((1, NSUBP), lambda g: (0, 0))],
        out_specs=pl.BlockSpec((1, SUBR, _B), lambda g: (g, 0, 0)),
        out_shape=jax.ShapeDtypeStruct((NCH2, SUBR, _B), jnp.int32),
    )(dst_s3, delta_t).reshape(E2)[:E]

    init = jnp.concatenate(
        [jnp.zeros((E_pad, 1), jnp.int32),
         jnp.full((E_pad, 1), -1, jnp.int32)], axis=1)
    pair = init.at[position].set(
        jnp.stack([src_s, dst_s], axis=1), mode='drop')
    src_pad = pair[:, 0]
    dstp = pair[:, 1]
    dl = jnp.where(dstp >= 0, dstp & (_W - 1), -1).astype(jnp.int32)
    dl3 = dl.reshape(NCHUNK, 1, _B)

    sf = jnp.nan_to_num(scalar_features)
    vf = jnp.nan_to_num(vector_features)
    x = jnp.concatenate([sf, vf.reshape(vf.shape[0], -1)], axis=-1)
    x = jnp.pad(x, ((0, N_pad - N), (0, 0)))

    pos4 = jnp.pad(pos, ((0, N_pad - N), (0, 1)))
    h0 = _matmul(x, params["W_pre_0"])
    table0 = jnp.concatenate(
        [h0, pos4, jnp.zeros((N_pad, 60), jnp.float32)], axis=1)

    feat = None
    for i in range(_NLAYERS):
        if i == 0:
            g = _sc_gather(table0, src_pad)
        else:
            g = _sc_gather(jnp.pad(h, ((0, 0), (0, 64))), src_pad)
        b1 = params[f"b1_{i}"].reshape(1, _C)
        common_specs = [
            pl.BlockSpec((1, 1, _B), lambda j, b, f: (j, 0, 0)),
            pl.BlockSpec((_NB, _C), lambda j, b, f: (0, 0)),
            pl.BlockSpec((1, _C), lambda j, b, f: (0, 0)),
            pl.BlockSpec((_C, _C), lambda j, b, f: (0, 0)),
        ]
        agg_spec = pl.BlockSpec((_W, _NSH * _C), lambda j, b, f: (b[j], 0))
        wargs = (params[f"W1_{i}"].astype(jnp.bfloat16), b1,
                 params[f"W2_{i}"].astype(jnp.bfloat16))
        if i == 0:
            agg, feat = pl.pallas_call(
                _edge_geom_agg_body,
                grid_spec=pltpu.PrefetchScalarGridSpec(
                    num_scalar_prefetch=2,
                    grid=(NCHUNK,),
                    in_specs=[
                        pl.BlockSpec((_B, 128), lambda j, b, f: (j, 0)),
                        pl.BlockSpec((_W, 4), lambda j, b, f: (b[j], 0)),
                    ] + common_specs,
                    out_specs=[agg_spec,
                               pl.BlockSpec((_B, 64), lambda j, b, f: (j, 0))],
                ),
                out_shape=[
                    jax.ShapeDtypeStruct((N_pad, _NSH * _C), jnp.float32),
                    jax.ShapeDtypeStruct((E_pad, 64), jnp.float32)],
            )(boc, first, g, pos4, dl3, *wargs)
        else:
            agg = pl.pallas_call(
                _edge_agg_body,
                grid_spec=pltpu.PrefetchScalarGridSpec(
                    num_scalar_prefetch=2,
                    grid=(NCHUNK,),
                    in_specs=[
                        pl.BlockSpec((_B, 64), lambda j, b, f: (j, 0)),
                        pl.BlockSpec((_B, 128), lambda j, b, f: (j, 0)),
                    ] + common_specs,
                    out_specs=agg_spec,
                ),
                out_shape=jax.ShapeDtypeStruct((N_pad, _NSH * _C),
                                               jnp.float32),
            )(boc, first, feat, g, dl3, *wargs)

        with_h = i + 1 < _NLAYERS
        d_in = x.shape[1]
        body = functools.partial(_node_out_body, with_h=with_h)
        in_specs = [
            pl.BlockSpec((_W, _NSH * _C), lambda t: (t, 0)),
            pl.BlockSpec((_W, d_in), lambda t: (t, 0)),
            pl.BlockSpec((_NSH * _C, _DIMH), lambda t: (0, 0)),
            pl.BlockSpec((d_in, _DIMH), lambda t: (0, 0)),
        ]
        operands = [agg, x, params[f"W_out_{i}"], params[f"W_self_{i}"]]
        if with_h:
            in_specs.append(pl.BlockSpec((_DIMH, _C), lambda t: (0, 0)))
            operands.append(params[f"W_pre_{i + 1}"])
            out_specs = [pl.BlockSpec((_W, _DIMH), lambda t: (t, 0)),
                         pl.BlockSpec((_W, _C), lambda t: (t, 0))]
            out_shape = [jax.ShapeDtypeStruct((N_pad, _DIMH), jnp.float32),
                         jax.ShapeDtypeStruct((N_pad, _C), jnp.float32)]
        else:
            out_specs = [pl.BlockSpec((_W, _DIMH), lambda t: (t, 0))]
            out_shape = [jax.ShapeDtypeStruct((N_pad, _DIMH), jnp.float32)]
        res = pl.pallas_call(
            body,
            grid=(N_pad // _W,),
            in_specs=in_specs,
            out_specs=out_specs,
            out_shape=out_shape,
        )(*operands)
        if with_h:
            x, h = res
        else:
            x = res[0]

    xo = x[:N]
    return (xo, xo[:, :_HS], xo[:, _HS:].reshape(N, _HV, 3))
